# baseline (speedup 1.0000x reference)
import jax
import jax.numpy as jnp
from jax.experimental import pallas as pl
from jax.experimental.pallas import tpu as pltpu

_LANE = 128
_SUBLANE = 8


def _rowdot_kernel(b_ref, x_ref, w_ref, o_ref):
    z = x_ref[...] * w_ref[...]
    o_ref[...] = jnp.sum(z, axis=1, keepdims=True) + b_ref[0, 0]


def _pick_block(n, candidates):
    for c in candidates:
        if n % c == 0:
            return c
    return _SUBLANE


def kernel(x, wt_padded, b_padded):
    B, F = x.shape
    dtype = x.dtype

    n_rows = B
    pad = (-n_rows) % _SUBLANE
    if pad:
        x = jnp.pad(x, ((0, pad), (0, 0)))
        B = x.shape[0]

    w2 = wt_padded[:, :1].reshape(1, F)
    b11 = b_padded[:1, :1]

    tb = _pick_block(B, (2048, 1024, 512, 256, 128, 64, 32, 16, 8))
    grid = (B // tb,)

    out = pl.pallas_call(
        _rowdot_kernel,
        out_shape=jax.ShapeDtypeStruct((B, 1), dtype),
        grid_spec=pl.GridSpec(
            grid=grid,
            in_specs=[
                pl.BlockSpec(memory_space=pltpu.SMEM),
                pl.BlockSpec((tb, F), lambda i: (i, 0)),
                pl.BlockSpec((1, F), lambda i: (0, 0)),
            ],
            out_specs=pl.BlockSpec((tb, 1), lambda i: (i, 0)),
        ),
        compiler_params=pltpu.CompilerParams(
            dimension_semantics=("parallel",),
        ),
        cost_estimate=pl.CostEstimate(
            flops=2 * B * F,
            transcendentals=0,
            bytes_accessed=B * F * 4 + F * 4 + B * 4,
        ),
    )(b11, x, w2)

    return out[:n_rows]


# speedup vs baseline: 2.2245x; 2.2245x over previous
import jax
import jax.numpy as jnp
from jax.experimental import pallas as pl
from jax.experimental.pallas import tpu as pltpu

_LANE = 128


def _rowdot_kernel(b_ref, x_ref, w_ref, o_ref):
    z = x_ref[...] * w_ref[...]
    o_ref[...] = jnp.sum(z, axis=2) + b_ref[0, 0]


def _pick_block(n, candidates):
    for c in candidates:
        if n % c == 0:
            return c
    return 1


def kernel(x, wt_padded, b_padded):
    B, F = x.shape
    dtype = x.dtype

    n_rows = B
    pad = (-n_rows) % _LANE
    if pad:
        x = jnp.pad(x, ((0, pad), (0, 0)))
        B = x.shape[0]

    s_total = B // _LANE
    x3 = x.reshape(s_total, _LANE, F)
    w3 = wt_padded[:, :1].reshape(1, 1, F)
    b11 = b_padded[:1, :1]

    s_blk = _pick_block(s_total, (32, 16, 8, 4, 2, 1))
    grid = (s_total // s_blk,)

    out = pl.pallas_call(
        _rowdot_kernel,
        out_shape=jax.ShapeDtypeStruct((s_total, _LANE), dtype),
        grid_spec=pl.GridSpec(
            grid=grid,
            in_specs=[
                pl.BlockSpec(memory_space=pltpu.SMEM),
                pl.BlockSpec((s_blk, _LANE, F), lambda i: (i, 0, 0)),
                pl.BlockSpec((1, 1, F), lambda i: (0, 0, 0)),
            ],
            out_specs=pl.BlockSpec((s_blk, _LANE), lambda i: (i, 0)),
        ),
        compiler_params=pltpu.CompilerParams(
            dimension_semantics=("parallel",),
        ),
        cost_estimate=pl.CostEstimate(
            flops=2 * B * F,
            transcendentals=0,
            bytes_accessed=B * F * 4 + F * 4 + B * 4,
        ),
    )(b11, x3, w3)

    return out.reshape(B, 1)[:n_rows]


# speedup vs baseline: 2.5108x; 1.1287x over previous
import jax
import jax.numpy as jnp
from jax.experimental import pallas as pl
from jax.experimental.pallas import tpu as pltpu

_LANE = 128


def _rowdot_kernel(b_ref, x_ref, w_ref, o_ref):
    z = x_ref[...] * w_ref[...]
    o_ref[...] = jnp.sum(z, axis=2) + b_ref[0, 0]


def _pick_block(n, candidates):
    for c in candidates:
        if n % c == 0:
            return c
    return 1


def kernel(x, wt_padded, b_padded):
    B, F = x.shape
    dtype = x.dtype

    n_rows = B
    pad = (-n_rows) % _LANE
    if pad:
        x = jnp.pad(x, ((0, pad), (0, 0)))
        B = x.shape[0]

    s_total = B // _LANE
    x3 = x.reshape(s_total, _LANE, F)
    w3 = wt_padded[:, :1].reshape(1, 1, F)
    b11 = b_padded[:1, :1]

    s_blk = _pick_block(s_total, (64, 32, 16, 8, 4, 2, 1))
    grid = (s_total // s_blk,)

    out = pl.pallas_call(
        _rowdot_kernel,
        out_shape=jax.ShapeDtypeStruct((s_total, _LANE), dtype),
        grid_spec=pl.GridSpec(
            grid=grid,
            in_specs=[
                pl.BlockSpec(memory_space=pltpu.SMEM),
                pl.BlockSpec((s_blk, _LANE, F), lambda i: (i, 0, 0)),
                pl.BlockSpec((1, 1, F), lambda i: (0, 0, 0)),
            ],
            out_specs=pl.BlockSpec((s_blk, _LANE), lambda i: (i, 0)),
        ),
        compiler_params=pltpu.CompilerParams(
            dimension_semantics=("parallel",),
        ),
        cost_estimate=pl.CostEstimate(
            flops=2 * B * F,
            transcendentals=0,
            bytes_accessed=B * F * 4 + F * 4 + B * 4,
        ),
    )(b11, x3, w3)

    return out.reshape(B, 1)[:n_rows]
